# baseline (speedup 1.0000x reference)
import functools

import jax
import jax.numpy as jnp
from jax import lax
from jax.experimental import pallas as pl
from jax.experimental.pallas import tpu as pltpu
from jax.experimental.pallas import tpu_sc as plsc

N = 10000
T = 12
D_IN = 128
H = 128
GATES = 4 * H

NC = 2
NS = 16
NW = NC * NS
L = 16

EDGE_BLK = 128
K = 16
E_IN = 320000
STAGES = -(-E_IN // (NW * K * EDGE_BLK))
E_PAD = NW * STAGES * K * EDGE_BLK
NBUF = 2
LAG = 1

N_ACC = 10240
DEG_W = 16

BN = 1000


def _vec_mesh():
    return plsc.VectorSubcoreMesh(core_axis_name="c", subcore_axis_name="s")


def _sc_degree(dst2d):
    @functools.partial(
        pl.kernel,
        out_type=jax.ShapeDtypeStruct((NC, N_ACC, DEG_W), jnp.float32),
        mesh=_vec_mesh(),
        scratch_types=[
            pltpu.VMEM((K, EDGE_BLK), jnp.int32),
            pltpu.VMEM((EDGE_BLK, DEG_W), jnp.float32),
            pltpu.VMEM_SHARED((N_ACC, DEG_W), jnp.float32),
        ],
    )
    def k(dst_hbm, out_hbm, dst_v, val_v, acc):
        c = lax.axis_index("c")
        s = lax.axis_index("s")
        w = c * NS + s
        zero = jnp.zeros((L,), dtype=jnp.float32)
        one = jnp.full((L,), 1.0, dtype=jnp.float32)

        @pl.loop(0, EDGE_BLK)
        def _(i):
            val_v[i, pl.ds(0, L)] = zero

        rows_pt = N_ACC // NS

        @pl.loop(0, rows_pt, step=EDGE_BLK)
        def _(r):
            pltpu.sync_copy(val_v, acc.at[pl.ds(s * rows_pt + r, EDGE_BLK)])

        @pl.loop(0, EDGE_BLK)
        def _(i):
            val_v[i, pl.ds(0, L)] = one

        plsc.subcore_barrier()

        @pl.loop(0, STAGES)
        def _(st):
            r0 = (w * STAGES + st) * K
            pltpu.sync_copy(dst_hbm.at[pl.ds(r0, K)], dst_v)
            for j in range(K):
                pltpu.sync_copy(val_v, acc.at[dst_v.at[j]], add=True)

        plsc.subcore_barrier()

        @pl.loop(0, rows_pt, step=EDGE_BLK)
        def _(r):
            b = s * rows_pt + r
            pltpu.sync_copy(acc.at[pl.ds(b, EDGE_BLK)],
                            out_hbm.at[c].at[pl.ds(b, EDGE_BLK)])

    return k(dst2d)


def _sc_scatter(z, src2d, dst2d):
    @functools.partial(
        pl.kernel,
        out_type=jax.ShapeDtypeStruct((NC, N_ACC, H), jnp.float32),
        mesh=_vec_mesh(),
        scratch_types=[
            pltpu.VMEM((K, EDGE_BLK), jnp.int32),
            pltpu.VMEM((K, EDGE_BLK), jnp.int32),
            [pltpu.VMEM((EDGE_BLK, H), jnp.float32)] * NBUF,
            pltpu.VMEM_SHARED((N_ACC, H), jnp.float32),
            [pltpu.SemaphoreType.DMA] * NBUF,
            [pltpu.SemaphoreType.DMA] * NBUF,
        ],
    )
    def k(z_hbm, src_hbm, dst_hbm, out_hbm, src_v, dst_v, rows, acc,
          gsem, ssem):
        c = lax.axis_index("c")
        s = lax.axis_index("s")
        w = (1 - c) * NS + s
        zero = jnp.zeros((L,), dtype=jnp.float32)

        @pl.loop(0, EDGE_BLK)
        def _(i):
            @pl.loop(0, H, step=L)
            def _(j):
                rows[0][i, pl.ds(j, L)] = zero

        rows_pt = N_ACC // NS

        @pl.loop(0, rows_pt, step=EDGE_BLK)
        def _(r):
            pltpu.sync_copy(rows[0], acc.at[pl.ds(s * rows_pt + r, EDGE_BLK)])

        plsc.subcore_barrier()

        @pl.loop(0, STAGES)
        def _(st):
            r0 = (w * STAGES + st) * K
            pltpu.sync_copy(src_hbm.at[pl.ds(r0, K)], src_v)
            pltpu.sync_copy(dst_hbm.at[pl.ds(r0, K)], dst_v)
            gd = [None] * NBUF
            sd = [None] * NBUF
            for g in range(K):
                b = g % NBUF
                if sd[b] is not None:
                    sd[b].wait()
                    sd[b] = None
                gd[b] = pltpu.async_copy(z_hbm.at[src_v.at[g]], rows[b],
                                         gsem[b])
                if g >= LAG:
                    gl = g - LAG
                    bl = gl % NBUF
                    gd[bl].wait()
                    sd[bl] = pltpu.async_copy(rows[bl], acc.at[dst_v.at[gl]],
                                              ssem[bl], add=True)
            for gl in range(K - LAG, K):
                bl = gl % NBUF
                gd[bl].wait()
                sd[bl] = pltpu.async_copy(rows[bl], acc.at[dst_v.at[gl]],
                                          ssem[bl], add=True)
            for d in sd:
                if d is not None:
                    d.wait()

        plsc.subcore_barrier()

        @pl.loop(0, rows_pt, step=EDGE_BLK)
        def _(r):
            b = s * rows_pt + r
            pltpu.sync_copy(acc.at[pl.ds(b, EDGE_BLK)],
                            out_hbm.at[c].at[pl.ds(b, EDGE_BLK)])

    return k(z, src2d, dst2d)


def _lstm_body(x_ref, wih_ref, whh_ref, bih_ref, bhh_ref, h_ref):
    b = bih_ref[...] + bhh_ref[...]
    h = jnp.zeros((BN, H), dtype=jnp.float32)
    c = jnp.zeros((BN, H), dtype=jnp.float32)
    for t in range(T):
        xt = x_ref[:, t, :]
        g = (jnp.dot(xt, wih_ref[...], preferred_element_type=jnp.float32)
             + jnp.dot(h, whh_ref[...], preferred_element_type=jnp.float32)
             + b)
        i = jax.nn.sigmoid(g[:, 0:H])
        f = jax.nn.sigmoid(g[:, H:2 * H])
        gg = jnp.tanh(g[:, 2 * H:3 * H])
        o = jax.nn.sigmoid(g[:, 3 * H:4 * H])
        c = f * c + i * gg
        h = o * jnp.tanh(c)
    h_ref[...] = h


def _lstm(xr, wihT, whhT, bih, bhh):
    return pl.pallas_call(
        _lstm_body,
        grid=(N // BN,),
        in_specs=[
            pl.BlockSpec((BN, T, D_IN), lambda i: (i, 0, 0)),
            pl.BlockSpec((D_IN, GATES), lambda i: (0, 0)),
            pl.BlockSpec((H, GATES), lambda i: (0, 0)),
            pl.BlockSpec((1, GATES), lambda i: (0, 0)),
            pl.BlockSpec((1, GATES), lambda i: (0, 0)),
        ],
        out_specs=pl.BlockSpec((BN, H), lambda i: (i, 0)),
        out_shape=jax.ShapeDtypeStruct((N, H), jnp.float32),
    )(xr, wihT, whhT, bih, bhh)


def _dinv(degp_ref):
    deg = degp_ref[0, :, 0:1] + degp_ref[1, :, 0:1] + 1.0
    return lax.rsqrt(deg)


def _prep_body(h_ref, w_ref, degp_ref, z_ref):
    dinv = _dinv(degp_ref)
    y = jnp.dot(h_ref[...], w_ref[...], preferred_element_type=jnp.float32)
    z_ref[...] = y * dinv


def _prep(h, W1, degp):
    return pl.pallas_call(
        _prep_body,
        grid=(N // BN,),
        in_specs=[
            pl.BlockSpec((BN, H), lambda i: (i, 0)),
            pl.BlockSpec((H, H), lambda i: (0, 0)),
            pl.BlockSpec((NC, BN, DEG_W), lambda i: (0, i, 0)),
        ],
        out_specs=pl.BlockSpec((BN, H), lambda i: (i, 0)),
        out_shape=jax.ShapeDtypeStruct((N, H), jnp.float32),
    )(h, W1, degp)


def _mid_body(s_ref, z_ref, degp_ref, b_ref, w_ref, out_ref):
    dinv = _dinv(degp_ref)
    t = s_ref[0] + s_ref[1] + z_ref[...]
    h1 = jnp.maximum(t * dinv + b_ref[...], 0.0)
    out_ref[...] = jnp.dot(
        h1, w_ref[...], preferred_element_type=jnp.float32) * dinv


def _mid(s1, z1, degp, b1, W2):
    return pl.pallas_call(
        _mid_body,
        grid=(N // BN,),
        in_specs=[
            pl.BlockSpec((NC, BN, H), lambda i: (0, i, 0)),
            pl.BlockSpec((BN, H), lambda i: (i, 0)),
            pl.BlockSpec((NC, BN, DEG_W), lambda i: (0, i, 0)),
            pl.BlockSpec((1, H), lambda i: (0, 0)),
            pl.BlockSpec((H, H), lambda i: (0, 0)),
        ],
        out_specs=pl.BlockSpec((BN, H), lambda i: (i, 0)),
        out_shape=jax.ShapeDtypeStruct((N, H), jnp.float32),
    )(s1, z1, degp, b1, W2)


def _fin_body(s_ref, z_ref, degp_ref, b_ref, out_ref):
    dinv = _dinv(degp_ref)
    t = s_ref[0] + s_ref[1] + z_ref[...]
    out_ref[...] = t * dinv + b_ref[...]


def _fin(s2, z2, degp, b2):
    return pl.pallas_call(
        _fin_body,
        grid=(N // BN,),
        in_specs=[
            pl.BlockSpec((NC, BN, H), lambda i: (0, i, 0)),
            pl.BlockSpec((BN, H), lambda i: (i, 0)),
            pl.BlockSpec((NC, BN, DEG_W), lambda i: (0, i, 0)),
            pl.BlockSpec((1, H), lambda i: (0, 0)),
        ],
        out_specs=pl.BlockSpec((BN, H), lambda i: (i, 0)),
        out_shape=jax.ShapeDtypeStruct((N, H), jnp.float32),
    )(s2, z2, degp, b2)


def kernel(x, edge_index, w_ih, w_hh, b_ih, b_hh, W1, b1, W2, b2):
    wihT = w_ih.T
    whhT = w_hh.T
    bih = b_ih.reshape(1, GATES)
    bhh = b_hh.reshape(1, GATES)
    b1r = b1.reshape(1, H)
    b2r = b2.reshape(1, H)

    pad = E_PAD - E_IN
    pad_dst = N + (jnp.arange(pad, dtype=jnp.int32) % (N_ACC - N))
    src2d = jnp.concatenate(
        [edge_index[0], jnp.zeros((pad,), jnp.int32)]).reshape(-1, EDGE_BLK)
    dst2d = jnp.concatenate(
        [edge_index[1], pad_dst]).reshape(-1, EDGE_BLK)

    degp = _sc_degree(dst2d)
    h = _lstm(x, wihT, whhT, bih, bhh)

    z1 = _prep(h, W1, degp)
    s1 = _sc_scatter(z1, src2d, dst2d)
    z2 = _mid(s1, z1, degp, b1r, W2)
    s2 = _sc_scatter(z2, src2d, dst2d)
    out = _fin(s2, z2, degp, b2r)
    return out


# speedup vs baseline: 2.4024x; 2.4024x over previous
import functools

import jax
import jax.numpy as jnp
from jax import lax
from jax.experimental import pallas as pl
from jax.experimental.pallas import tpu as pltpu
from jax.experimental.pallas import tpu_sc as plsc

N = 10000
T = 12
D_IN = 128
H = 128
GATES = 4 * H

NC = 2
NS = 16
NW = NC * NS
L = 16

EDGE_BLK = 128
K = 16
E_IN = 320000
STAGES = -(-E_IN // (NW * K * EDGE_BLK))
E_PAD = NW * STAGES * K * EDGE_BLK
NBUF = 2
LAG = 1

N_ACC = 10240
DEG_W = 16

BN = 1000


def _vec_mesh():
    return plsc.VectorSubcoreMesh(core_axis_name="c", subcore_axis_name="s")


def _sc_degree(dst2d):
    @functools.partial(
        pl.kernel,
        out_type=jax.ShapeDtypeStruct((NC, N_ACC, DEG_W), jnp.float32),
        mesh=_vec_mesh(),
        scratch_types=[
            pltpu.VMEM((K, EDGE_BLK), jnp.int32),
            pltpu.VMEM((EDGE_BLK, DEG_W), jnp.float32),
            pltpu.VMEM_SHARED((N_ACC, DEG_W), jnp.float32),
        ],
    )
    def k(dst_hbm, out_hbm, dst_v, val_v, acc):
        c = lax.axis_index("c")
        s = lax.axis_index("s")
        w = c * NS + s
        zero = jnp.zeros((L,), dtype=jnp.float32)
        one = jnp.full((L,), 1.0, dtype=jnp.float32)

        @pl.loop(0, EDGE_BLK)
        def _(i):
            val_v[i, pl.ds(0, L)] = zero

        rows_pt = N_ACC // NS

        @pl.loop(0, rows_pt, step=EDGE_BLK)
        def _(r):
            pltpu.sync_copy(val_v, acc.at[pl.ds(s * rows_pt + r, EDGE_BLK)])

        @pl.loop(0, EDGE_BLK)
        def _(i):
            val_v[i, pl.ds(0, L)] = one

        plsc.subcore_barrier()

        @pl.loop(0, STAGES)
        def _(st):
            r0 = (w * STAGES + st) * K
            pltpu.sync_copy(dst_hbm.at[pl.ds(r0, K)], dst_v)
            for j in range(K):
                pltpu.sync_copy(val_v, acc.at[dst_v.at[j]], add=True)

        plsc.subcore_barrier()

        @pl.loop(0, rows_pt, step=EDGE_BLK)
        def _(r):
            b = s * rows_pt + r
            pltpu.sync_copy(acc.at[pl.ds(b, EDGE_BLK)],
                            out_hbm.at[c].at[pl.ds(b, EDGE_BLK)])

    return k(dst2d)


def _sc_scatter(z, src2d, dst2d):
    @functools.partial(
        pl.kernel,
        out_type=jax.ShapeDtypeStruct((NC, N_ACC, H), jnp.float32),
        mesh=_vec_mesh(),
        scratch_types=[
            pltpu.VMEM((K, EDGE_BLK), jnp.int32),
            pltpu.VMEM((K, EDGE_BLK), jnp.int32),
            [pltpu.VMEM((EDGE_BLK, H), jnp.float32)] * NBUF,
            pltpu.VMEM_SHARED((N_ACC, H), jnp.float32),
            [pltpu.SemaphoreType.DMA] * NBUF,
            [pltpu.SemaphoreType.DMA] * NBUF,
        ],
    )
    def k(z_hbm, src_hbm, dst_hbm, out_hbm, src_v, dst_v, rows, acc,
          gsem, ssem):
        c = lax.axis_index("c")
        s = lax.axis_index("s")
        w = c * NS + s
        zero = jnp.zeros((L,), dtype=jnp.float32)

        @pl.loop(0, EDGE_BLK)
        def _(i):
            @pl.loop(0, H, step=L)
            def _(j):
                rows[0][i, pl.ds(j, L)] = zero

        rows_pt = N_ACC // NS

        @pl.loop(0, rows_pt, step=EDGE_BLK)
        def _(r):
            pltpu.sync_copy(rows[0], acc.at[pl.ds(s * rows_pt + r, EDGE_BLK)])

        plsc.subcore_barrier()

        @pl.loop(0, STAGES)
        def _(st):
            r0 = (w * STAGES + st) * K
            pltpu.sync_copy(src_hbm.at[pl.ds(r0, K)], src_v)
            pltpu.sync_copy(dst_hbm.at[pl.ds(r0, K)], dst_v)
            gd = [None] * NBUF
            sd = [None] * NBUF
            for g in range(K):
                b = g % NBUF
                if sd[b] is not None:
                    sd[b].wait()
                    sd[b] = None
                gd[b] = pltpu.async_copy(z_hbm.at[src_v.at[g]], rows[b],
                                         gsem[b])
                if g >= LAG:
                    gl = g - LAG
                    bl = gl % NBUF
                    gd[bl].wait()
                    sd[bl] = pltpu.async_copy(rows[bl], acc.at[dst_v.at[gl]],
                                              ssem[bl], add=True)
            for gl in range(K - LAG, K):
                bl = gl % NBUF
                gd[bl].wait()
                sd[bl] = pltpu.async_copy(rows[bl], acc.at[dst_v.at[gl]],
                                          ssem[bl], add=True)
            for d in sd:
                if d is not None:
                    d.wait()

        plsc.subcore_barrier()

        @pl.loop(0, rows_pt, step=EDGE_BLK)
        def _(r):
            b = s * rows_pt + r
            pltpu.sync_copy(acc.at[pl.ds(b, EDGE_BLK)],
                            out_hbm.at[c].at[pl.ds(b, EDGE_BLK)])

    return k(z, src2d, dst2d)


def _lstm_body(x_ref, wih_ref, whh_ref, bih_ref, bhh_ref, h_ref):
    b = bih_ref[...] + bhh_ref[...]
    h = jnp.zeros((BN, H), dtype=jnp.float32)
    c = jnp.zeros((BN, H), dtype=jnp.float32)
    for t in range(T):
        xt = x_ref[:, t, :]
        g = (jnp.dot(xt, wih_ref[...], preferred_element_type=jnp.float32)
             + jnp.dot(h, whh_ref[...], preferred_element_type=jnp.float32)
             + b)
        i = jax.nn.sigmoid(g[:, 0:H])
        f = jax.nn.sigmoid(g[:, H:2 * H])
        gg = jnp.tanh(g[:, 2 * H:3 * H])
        o = jax.nn.sigmoid(g[:, 3 * H:4 * H])
        c = f * c + i * gg
        h = o * jnp.tanh(c)
    h_ref[...] = h


def _lstm(xr, wihT, whhT, bih, bhh):
    return pl.pallas_call(
        _lstm_body,
        grid=(N // BN,),
        in_specs=[
            pl.BlockSpec((BN, T, D_IN), lambda i: (i, 0, 0)),
            pl.BlockSpec((D_IN, GATES), lambda i: (0, 0)),
            pl.BlockSpec((H, GATES), lambda i: (0, 0)),
            pl.BlockSpec((1, GATES), lambda i: (0, 0)),
            pl.BlockSpec((1, GATES), lambda i: (0, 0)),
        ],
        out_specs=pl.BlockSpec((BN, H), lambda i: (i, 0)),
        out_shape=jax.ShapeDtypeStruct((N, H), jnp.float32),
    )(xr, wihT, whhT, bih, bhh)


def _dinv(degp_ref):
    deg = degp_ref[0, :, 0:1] + degp_ref[1, :, 0:1] + 1.0
    return lax.rsqrt(deg)


def _prep_body(h_ref, w_ref, degp_ref, z_ref):
    dinv = _dinv(degp_ref)
    y = jnp.dot(h_ref[...], w_ref[...], preferred_element_type=jnp.float32)
    z_ref[...] = y * dinv


def _prep(h, W1, degp):
    return pl.pallas_call(
        _prep_body,
        grid=(N // BN,),
        in_specs=[
            pl.BlockSpec((BN, H), lambda i: (i, 0)),
            pl.BlockSpec((H, H), lambda i: (0, 0)),
            pl.BlockSpec((NC, BN, DEG_W), lambda i: (0, i, 0)),
        ],
        out_specs=pl.BlockSpec((BN, H), lambda i: (i, 0)),
        out_shape=jax.ShapeDtypeStruct((N, H), jnp.float32),
    )(h, W1, degp)


def _mid_body(s_ref, z_ref, degp_ref, b_ref, w_ref, out_ref):
    dinv = _dinv(degp_ref)
    t = s_ref[0] + s_ref[1] + z_ref[...]
    h1 = jnp.maximum(t * dinv + b_ref[...], 0.0)
    out_ref[...] = jnp.dot(
        h1, w_ref[...], preferred_element_type=jnp.float32) * dinv


def _mid(s1, z1, degp, b1, W2):
    return pl.pallas_call(
        _mid_body,
        grid=(N // BN,),
        in_specs=[
            pl.BlockSpec((NC, BN, H), lambda i: (0, i, 0)),
            pl.BlockSpec((BN, H), lambda i: (i, 0)),
            pl.BlockSpec((NC, BN, DEG_W), lambda i: (0, i, 0)),
            pl.BlockSpec((1, H), lambda i: (0, 0)),
            pl.BlockSpec((H, H), lambda i: (0, 0)),
        ],
        out_specs=pl.BlockSpec((BN, H), lambda i: (i, 0)),
        out_shape=jax.ShapeDtypeStruct((N, H), jnp.float32),
    )(s1, z1, degp, b1, W2)


def _fin_body(s_ref, z_ref, degp_ref, b_ref, out_ref):
    dinv = _dinv(degp_ref)
    t = s_ref[0] + s_ref[1] + z_ref[...]
    out_ref[...] = t * dinv + b_ref[...]


def _fin(s2, z2, degp, b2):
    return pl.pallas_call(
        _fin_body,
        grid=(N // BN,),
        in_specs=[
            pl.BlockSpec((NC, BN, H), lambda i: (0, i, 0)),
            pl.BlockSpec((BN, H), lambda i: (i, 0)),
            pl.BlockSpec((NC, BN, DEG_W), lambda i: (0, i, 0)),
            pl.BlockSpec((1, H), lambda i: (0, 0)),
        ],
        out_specs=pl.BlockSpec((BN, H), lambda i: (i, 0)),
        out_shape=jax.ShapeDtypeStruct((N, H), jnp.float32),
    )(s2, z2, degp, b2)


def kernel(x, edge_index, w_ih, w_hh, b_ih, b_hh, W1, b1, W2, b2):
    wihT = w_ih.T
    whhT = w_hh.T
    bih = b_ih.reshape(1, GATES)
    bhh = b_hh.reshape(1, GATES)
    b1r = b1.reshape(1, H)
    b2r = b2.reshape(1, H)

    pad = E_PAD - E_IN
    pad_dst = N + (jnp.arange(pad, dtype=jnp.int32) % (N_ACC - N))
    pad_src = jnp.arange(pad, dtype=jnp.int32) % N
    src2d = jnp.concatenate(
        [edge_index[0], pad_src]).reshape(-1, EDGE_BLK)
    dst2d = jnp.concatenate(
        [edge_index[1], pad_dst]).reshape(-1, EDGE_BLK)

    degp = _sc_degree(dst2d)
    h = _lstm(x, wihT, whhT, bih, bhh)

    z1 = _prep(h, W1, degp)
    s1 = _sc_scatter(z1, src2d, dst2d)
    z2 = _mid(s1, z1, degp, b1r, W2)
    s2 = _sc_scatter(z2, src2d, dst2d)
    out = _fin(s2, z2, degp, b2r)
    return out


# speedup vs baseline: 2.4707x; 1.0284x over previous
import functools

import jax
import jax.numpy as jnp
from jax import lax
from jax.experimental import pallas as pl
from jax.experimental.pallas import tpu as pltpu
from jax.experimental.pallas import tpu_sc as plsc

N = 10000
T = 12
D_IN = 128
H = 128
GATES = 4 * H

NC = 2
NS = 16
NW = NC * NS
L = 16

EDGE_BLK = 128
K = 16
E_IN = 320000
STAGES = -(-E_IN // (NW * K * EDGE_BLK))
E_PAD = NW * STAGES * K * EDGE_BLK
NBUF = 2
LAG = 1

N_ACC = 10240
DEG_W = 128

BN = 1000


def _vec_mesh():
    return plsc.VectorSubcoreMesh(core_axis_name="c", subcore_axis_name="s")


def _sc_degree(dst2d):
    @functools.partial(
        pl.kernel,
        out_type=jax.ShapeDtypeStruct((NC, N_ACC, DEG_W), jnp.float32),
        mesh=_vec_mesh(),
        scratch_types=[
            pltpu.VMEM((K, EDGE_BLK), jnp.int32),
            pltpu.VMEM((EDGE_BLK, DEG_W), jnp.float32),
            pltpu.VMEM_SHARED((N_ACC, DEG_W), jnp.float32),
        ],
    )
    def k(dst_hbm, out_hbm, dst_v, val_v, acc):
        c = lax.axis_index("c")
        s = lax.axis_index("s")
        w = c * NS + s
        zero = jnp.zeros((L,), dtype=jnp.float32)
        one = jnp.full((L,), 1.0, dtype=jnp.float32)

        @pl.loop(0, EDGE_BLK)
        def _(i):
            @pl.loop(0, DEG_W, step=L)
            def _(j):
                val_v[i, pl.ds(j, L)] = zero

        rows_pt = N_ACC // NS

        @pl.loop(0, rows_pt, step=EDGE_BLK)
        def _(r):
            pltpu.sync_copy(val_v, acc.at[pl.ds(s * rows_pt + r, EDGE_BLK)])

        @pl.loop(0, EDGE_BLK)
        def _(i):
            @pl.loop(0, DEG_W, step=L)
            def _(j):
                val_v[i, pl.ds(j, L)] = one

        plsc.subcore_barrier()

        @pl.loop(0, STAGES)
        def _(st):
            r0 = (w * STAGES + st) * K
            pltpu.sync_copy(dst_hbm.at[pl.ds(r0, K)], dst_v)
            for j in range(K):
                pltpu.sync_copy(val_v, acc.at[dst_v.at[j]], add=True)

        plsc.subcore_barrier()

        @pl.loop(0, rows_pt, step=EDGE_BLK)
        def _(r):
            b = s * rows_pt + r
            pltpu.sync_copy(acc.at[pl.ds(b, EDGE_BLK)],
                            out_hbm.at[c].at[pl.ds(b, EDGE_BLK)])

    return k(dst2d)


def _sc_scatter(z, src2d, dst2d):
    @functools.partial(
        pl.kernel,
        out_type=jax.ShapeDtypeStruct((NC, N_ACC, H), jnp.float32),
        mesh=_vec_mesh(),
        scratch_types=[
            pltpu.VMEM((K, EDGE_BLK), jnp.int32),
            pltpu.VMEM((K, EDGE_BLK), jnp.int32),
            [pltpu.VMEM((EDGE_BLK, H), jnp.float32)] * NBUF,
            pltpu.VMEM_SHARED((N_ACC, H), jnp.float32),
            [pltpu.SemaphoreType.DMA] * NBUF,
            [pltpu.SemaphoreType.DMA] * NBUF,
        ],
    )
    def k(z_hbm, src_hbm, dst_hbm, out_hbm, src_v, dst_v, rows, acc,
          gsem, ssem):
        c = lax.axis_index("c")
        s = lax.axis_index("s")
        w = c * NS + s
        zero = jnp.zeros((L,), dtype=jnp.float32)

        @pl.loop(0, EDGE_BLK)
        def _(i):
            @pl.loop(0, H, step=L)
            def _(j):
                rows[0][i, pl.ds(j, L)] = zero

        rows_pt = N_ACC // NS

        @pl.loop(0, rows_pt, step=EDGE_BLK)
        def _(r):
            pltpu.sync_copy(rows[0], acc.at[pl.ds(s * rows_pt + r, EDGE_BLK)])

        plsc.subcore_barrier()

        @pl.loop(0, STAGES)
        def _(st):
            r0 = (w * STAGES + st) * K
            pltpu.sync_copy(src_hbm.at[pl.ds(r0, K)], src_v)
            pltpu.sync_copy(dst_hbm.at[pl.ds(r0, K)], dst_v)
            gd = [None] * NBUF
            for g in range(K):
                b = g % NBUF
                gd[b] = pltpu.async_copy(z_hbm.at[src_v.at[g]], rows[b],
                                         gsem[b])
                if g >= LAG:
                    gl = g - LAG
                    bl = gl % NBUF
                    gd[bl].wait()
                    pltpu.sync_copy(rows[bl], acc.at[dst_v.at[gl]], add=True)
            for gl in range(K - LAG, K):
                bl = gl % NBUF
                gd[bl].wait()
                pltpu.sync_copy(rows[bl], acc.at[dst_v.at[gl]], add=True)

        plsc.subcore_barrier()

        @pl.loop(0, rows_pt, step=EDGE_BLK)
        def _(r):
            b = s * rows_pt + r
            pltpu.sync_copy(acc.at[pl.ds(b, EDGE_BLK)],
                            out_hbm.at[c].at[pl.ds(b, EDGE_BLK)])

    return k(z, src2d, dst2d)


def _eprep_body(ei_ref, src_ref, dst_ref):
    srcf = ei_ref[0].reshape(E_IN // EDGE_BLK, EDGE_BLK)
    dstf = ei_ref[1].reshape(E_IN // EDGE_BLK, EDGE_BLK)
    prows = (E_PAD - E_IN) // EDGE_BLK
    padi = (lax.broadcasted_iota(jnp.int32, (prows, EDGE_BLK), 0) * EDGE_BLK
            + lax.broadcasted_iota(jnp.int32, (prows, EDGE_BLK), 1))
    src_ref[...] = jnp.concatenate([srcf, padi % N], axis=0)
    dst_ref[...] = jnp.concatenate([dstf, N + padi % (N_ACC - N)], axis=0)


def _eprep(edge_index):
    rows = E_PAD // EDGE_BLK
    return pl.pallas_call(
        _eprep_body,
        in_specs=[pl.BlockSpec((2, E_IN), lambda: (0, 0))],
        out_specs=[pl.BlockSpec((rows, EDGE_BLK), lambda: (0, 0)),
                   pl.BlockSpec((rows, EDGE_BLK), lambda: (0, 0))],
        out_shape=[jax.ShapeDtypeStruct((rows, EDGE_BLK), jnp.int32),
                   jax.ShapeDtypeStruct((rows, EDGE_BLK), jnp.int32)],
    )(edge_index)


def _lstm_body(x_ref, wih_ref, whh_ref, bih_ref, bhh_ref, h_ref):
    b = bih_ref[...] + bhh_ref[...]
    h = jnp.zeros((BN, H), dtype=jnp.float32)
    c = jnp.zeros((BN, H), dtype=jnp.float32)
    for t in range(T):
        xt = x_ref[:, t, :]
        g = (jnp.dot(xt, wih_ref[...], preferred_element_type=jnp.float32)
             + jnp.dot(h, whh_ref[...], preferred_element_type=jnp.float32)
             + b)
        i = jax.nn.sigmoid(g[:, 0:H])
        f = jax.nn.sigmoid(g[:, H:2 * H])
        gg = jnp.tanh(g[:, 2 * H:3 * H])
        o = jax.nn.sigmoid(g[:, 3 * H:4 * H])
        c = f * c + i * gg
        h = o * jnp.tanh(c)
    h_ref[...] = h


def _lstm(xr, wihT, whhT, bih, bhh):
    return pl.pallas_call(
        _lstm_body,
        grid=(N // BN,),
        in_specs=[
            pl.BlockSpec((BN, T, D_IN), lambda i: (i, 0, 0)),
            pl.BlockSpec((D_IN, GATES), lambda i: (0, 0)),
            pl.BlockSpec((H, GATES), lambda i: (0, 0)),
            pl.BlockSpec((1, GATES), lambda i: (0, 0)),
            pl.BlockSpec((1, GATES), lambda i: (0, 0)),
        ],
        out_specs=pl.BlockSpec((BN, H), lambda i: (i, 0)),
        out_shape=jax.ShapeDtypeStruct((N, H), jnp.float32),
    )(xr, wihT, whhT, bih, bhh)


def _dinv(degp_ref):
    deg = degp_ref[0, :, 0:1] + degp_ref[1, :, 0:1] + 1.0
    return lax.rsqrt(deg)


def _prep_body(h_ref, w_ref, degp_ref, z_ref):
    dinv = _dinv(degp_ref)
    y = jnp.dot(h_ref[...], w_ref[...], preferred_element_type=jnp.float32)
    z_ref[...] = y * dinv


def _prep(h, W1, degp):
    return pl.pallas_call(
        _prep_body,
        grid=(N // BN,),
        in_specs=[
            pl.BlockSpec((BN, H), lambda i: (i, 0)),
            pl.BlockSpec((H, H), lambda i: (0, 0)),
            pl.BlockSpec((NC, BN, DEG_W), lambda i: (0, i, 0)),
        ],
        out_specs=pl.BlockSpec((BN, H), lambda i: (i, 0)),
        out_shape=jax.ShapeDtypeStruct((N, H), jnp.float32),
    )(h, W1, degp)


def _mid_body(s_ref, z_ref, degp_ref, b_ref, w_ref, out_ref):
    dinv = _dinv(degp_ref)
    t = s_ref[0] + s_ref[1] + z_ref[...]
    h1 = jnp.maximum(t * dinv + b_ref[...], 0.0)
    out_ref[...] = jnp.dot(
        h1, w_ref[...], preferred_element_type=jnp.float32) * dinv


def _mid(s1, z1, degp, b1, W2):
    return pl.pallas_call(
        _mid_body,
        grid=(N // BN,),
        in_specs=[
            pl.BlockSpec((NC, BN, H), lambda i: (0, i, 0)),
            pl.BlockSpec((BN, H), lambda i: (i, 0)),
            pl.BlockSpec((NC, BN, DEG_W), lambda i: (0, i, 0)),
            pl.BlockSpec((1, H), lambda i: (0, 0)),
            pl.BlockSpec((H, H), lambda i: (0, 0)),
        ],
        out_specs=pl.BlockSpec((BN, H), lambda i: (i, 0)),
        out_shape=jax.ShapeDtypeStruct((N, H), jnp.float32),
    )(s1, z1, degp, b1, W2)


def _fin_body(s_ref, z_ref, degp_ref, b_ref, out_ref):
    dinv = _dinv(degp_ref)
    t = s_ref[0] + s_ref[1] + z_ref[...]
    out_ref[...] = t * dinv + b_ref[...]


def _fin(s2, z2, degp, b2):
    return pl.pallas_call(
        _fin_body,
        grid=(N // BN,),
        in_specs=[
            pl.BlockSpec((NC, BN, H), lambda i: (0, i, 0)),
            pl.BlockSpec((BN, H), lambda i: (i, 0)),
            pl.BlockSpec((NC, BN, DEG_W), lambda i: (0, i, 0)),
            pl.BlockSpec((1, H), lambda i: (0, 0)),
        ],
        out_specs=pl.BlockSpec((BN, H), lambda i: (i, 0)),
        out_shape=jax.ShapeDtypeStruct((N, H), jnp.float32),
    )(s2, z2, degp, b2)


def kernel(x, edge_index, w_ih, w_hh, b_ih, b_hh, W1, b1, W2, b2):
    wihT = w_ih.T
    whhT = w_hh.T
    bih = b_ih.reshape(1, GATES)
    bhh = b_hh.reshape(1, GATES)
    b1r = b1.reshape(1, H)
    b2r = b2.reshape(1, H)

    src2d, dst2d = _eprep(edge_index)

    degp = _sc_degree(dst2d)
    h = _lstm(x, wihT, whhT, bih, bhh)

    z1 = _prep(h, W1, degp)
    s1 = _sc_scatter(z1, src2d, dst2d)
    z2 = _mid(s1, z1, degp, b1r, W2)
    s2 = _sc_scatter(z2, src2d, dst2d)
    out = _fin(s2, z2, degp, b2r)
    return out
